# baseline (speedup 1.0000x reference)
import functools
import math

import jax
import jax.numpy as jnp
from jax.experimental import pallas as pl
from jax.experimental.pallas import tpu as pltpu


def _fused_kernel(x_ref, wq_ref, wk_ref, wv_ref, o_ref,
                  xb_ref, ta_ref, q_ref, k_ref, v_ref,
                  *, B, T, N, H, d_head, n_k, scale):
    D = x_ref.shape[3]
    R = B * T * N
    S = B * N
    C = q_ref.shape[1]
    HH = C // d_head
    KT = wq_ref.shape[0]
    j = pl.program_id(0)
    i = pl.program_id(1)

    @pl.when(jnp.logical_and(j == 0, i == 0))
    def _pool():
        x = x_ref[...]
        m = jnp.max(x, axis=1, keepdims=True)
        e = jnp.exp(x - m)
        ta = jnp.sum(x * e, axis=1) / jnp.sum(e, axis=1)
        ta_ref[...] = ta.reshape(S, D).astype(jnp.bfloat16)
        xb_ref[...] = x.reshape(R, D).astype(jnp.bfloat16)

    row = pl.multiple_of(i * KT, KT)
    xs = xb_ref[:, pl.ds(row, KT)]
    ts = ta_ref[:, pl.ds(row, KT)]
    wq = wq_ref[...].astype(jnp.bfloat16)
    wk = wk_ref[...].astype(jnp.bfloat16)
    wv = wv_ref[...].astype(jnp.bfloat16)
    pq = jnp.dot(xs, wq, preferred_element_type=jnp.float32)
    pk = jnp.dot(ts, wk, preferred_element_type=jnp.float32)
    pv = jnp.dot(ts, wv, preferred_element_type=jnp.float32)

    @pl.when(i == 0)
    def _init():
        q_ref[...] = pq
        k_ref[...] = pk
        v_ref[...] = pv

    @pl.when(i > 0)
    def _acc():
        q_ref[...] += pq
        k_ref[...] += pk
        v_ref[...] += pv

    @pl.when(i == n_k - 1)
    def _attn():
        q = (q_ref[...] * scale).astype(jnp.bfloat16)
        k = k_ref[...]
        v = v_ref[...]
        CC = B * HH * N
        rh = (jax.lax.broadcasted_iota(jnp.int32, (CC, C), 0) % (HH * N)) // N
        lh = jax.lax.broadcasted_iota(jnp.int32, (CC, C), 1) // d_head
        hmask = rh == lh
        kb = jnp.broadcast_to(
            k.reshape(B, 1, N, C), (B, HH, N, C)).reshape(CC, C)
        vb = jnp.broadcast_to(
            v.reshape(B, 1, N, C), (B, HH, N, C)).reshape(CC, C)
        zero = jnp.zeros((), jnp.bfloat16)
        kbig = jnp.where(hmask, kb.astype(jnp.bfloat16), zero)
        vbig = jnp.where(hmask, vb.astype(jnp.bfloat16), zero)

        s = jax.lax.dot_general(q, kbig, (((1,), (1,)), ((), ())),
                                preferred_element_type=jnp.float32)
        rb = jax.lax.broadcasted_iota(jnp.int32, (R, CC), 0) // (T * N)
        cb = jax.lax.broadcasted_iota(jnp.int32, (R, CC), 1) // (HH * N)
        p = jnp.where(rb == cb, jnp.maximum(s, 0.0),
                      0.0).astype(jnp.bfloat16)

        o = jnp.dot(p, vbig, preferred_element_type=jnp.float32)
        o = o.reshape(B, T, N, C) + v.reshape(B, 1, N, C)
        o_ref[...] = o.astype(o_ref.dtype)


def kernel(x, boxes_in_flat, wq, wk, wv):
    del boxes_in_flat
    B, T, N, D = x.shape
    H = 8
    d_head = D // H
    R = B * T * N
    S = B * N
    scale = 1.0 / math.sqrt(d_head)

    NJ = 2
    C = D // NJ
    KT = 128
    n_k = D // KT

    kern = functools.partial(
        _fused_kernel, B=B, T=T, N=N, H=H, d_head=d_head, n_k=n_k,
        scale=scale)
    return pl.pallas_call(
        kern,
        out_shape=jax.ShapeDtypeStruct((B, T, N, D), x.dtype),
        grid=(NJ, n_k),
        in_specs=[
            pl.BlockSpec((B, T, N, D), lambda j, i: (0, 0, 0, 0)),
            pl.BlockSpec((KT, C), lambda j, i: (i, j)),
            pl.BlockSpec((KT, C), lambda j, i: (i, j)),
            pl.BlockSpec((KT, C), lambda j, i: (i, j)),
        ],
        out_specs=pl.BlockSpec((B, T, N, C), lambda j, i: (0, 0, 0, j)),
        scratch_shapes=[
            pltpu.VMEM((R, D), jnp.bfloat16),
            pltpu.VMEM((S, D), jnp.bfloat16),
            pltpu.VMEM((R, C), jnp.float32),
            pltpu.VMEM((S, C), jnp.float32),
            pltpu.VMEM((S, C), jnp.float32),
        ],
        compiler_params=pltpu.CompilerParams(
            dimension_semantics=("arbitrary", "arbitrary")),
    )(x, wq, wk, wv)


# speedup vs baseline: 1.6684x; 1.6684x over previous
import functools
import math

import jax
import jax.numpy as jnp
from jax.experimental import pallas as pl
from jax.experimental.pallas import tpu as pltpu


def _fused_kernel(x_ref, wq_ref, wk_ref, wv_ref, o_ref,
                  xb_ref, ta_ref, q_ref, k_ref, v_ref,
                  *, B, T, N, H, d_head, n_k, scale):
    D = x_ref.shape[3]
    R = B * T * N
    S = B * N
    KT = wq_ref.shape[0]
    i = pl.program_id(0)

    @pl.when(i == 0)
    def _pool():
        x = x_ref[...]
        m = jnp.max(x, axis=1, keepdims=True)
        e = jnp.exp(x - m)
        ta = jnp.sum(x * e, axis=1) / jnp.sum(e, axis=1)
        ta_ref[...] = ta.reshape(S, D).astype(jnp.bfloat16)
        xb_ref[...] = x.reshape(R, D).astype(jnp.bfloat16)

    row = pl.multiple_of(i * KT, KT)
    xs = xb_ref[:, pl.ds(row, KT)]
    ts = ta_ref[:, pl.ds(row, KT)]
    wq = wq_ref[...].astype(jnp.bfloat16)
    wk = wk_ref[...].astype(jnp.bfloat16)
    wv = wv_ref[...].astype(jnp.bfloat16)
    pq = jnp.dot(xs, wq, preferred_element_type=jnp.float32)
    pk = jnp.dot(ts, wk, preferred_element_type=jnp.float32)
    pv = jnp.dot(ts, wv, preferred_element_type=jnp.float32)

    @pl.when(i == 0)
    def _init():
        q_ref[...] = pq
        k_ref[...] = pk
        v_ref[...] = pv

    @pl.when(i > 0)
    def _acc():
        q_ref[...] += pq
        k_ref[...] += pk
        v_ref[...] += pv

    @pl.when(i == n_k - 1)
    def _attn():
        q = (q_ref[...] * scale).astype(jnp.bfloat16)
        k = k_ref[...]
        v = v_ref[...]
        C = B * H * N
        rh = (jax.lax.broadcasted_iota(jnp.int32, (C, D), 0) % (H * N)) // N
        lh = jax.lax.broadcasted_iota(jnp.int32, (C, D), 1) // d_head
        hmask = rh == lh
        kb = jnp.broadcast_to(
            k.reshape(B, 1, N, D), (B, H, N, D)).reshape(C, D)
        vb = jnp.broadcast_to(
            v.reshape(B, 1, N, D), (B, H, N, D)).reshape(C, D)
        zero = jnp.zeros((), jnp.bfloat16)
        kbig = jnp.where(hmask, kb.astype(jnp.bfloat16), zero)
        vbig = jnp.where(hmask, vb.astype(jnp.bfloat16), zero)

        s = jax.lax.dot_general(q, kbig, (((1,), (1,)), ((), ())),
                                preferred_element_type=jnp.float32)
        rb = jax.lax.broadcasted_iota(jnp.int32, (R, C), 0) // (T * N)
        cb = jax.lax.broadcasted_iota(jnp.int32, (R, C), 1) // (H * N)
        p = jnp.where(rb == cb, jnp.maximum(s, 0.0),
                      0.0).astype(jnp.bfloat16)

        o = jnp.dot(p, vbig, preferred_element_type=jnp.float32)
        o = o.reshape(B, T, N, D) + v.reshape(B, 1, N, D)
        o_ref[...] = o.astype(o_ref.dtype)


def kernel(x, boxes_in_flat, wq, wk, wv):
    del boxes_in_flat
    B, T, N, D = x.shape
    H = 8
    d_head = D // H
    R = B * T * N
    S = B * N
    scale = 1.0 / math.sqrt(d_head)

    KT = 256
    n_k = D // KT

    kern = functools.partial(
        _fused_kernel, B=B, T=T, N=N, H=H, d_head=d_head, n_k=n_k,
        scale=scale)
    return pl.pallas_call(
        kern,
        out_shape=jax.ShapeDtypeStruct((B, T, N, D), x.dtype),
        grid=(n_k,),
        in_specs=[
            pl.BlockSpec((B, T, N, D), lambda i: (0, 0, 0, 0)),
            pl.BlockSpec((KT, D), lambda i: (i, 0)),
            pl.BlockSpec((KT, D), lambda i: (i, 0)),
            pl.BlockSpec((KT, D), lambda i: (i, 0)),
        ],
        out_specs=pl.BlockSpec((B, T, N, D), lambda i: (0, 0, 0, 0)),
        scratch_shapes=[
            pltpu.VMEM((R, D), jnp.bfloat16),
            pltpu.VMEM((S, D), jnp.bfloat16),
            pltpu.VMEM((R, D), jnp.float32),
            pltpu.VMEM((S, D), jnp.float32),
            pltpu.VMEM((S, D), jnp.float32),
        ],
        compiler_params=pltpu.CompilerParams(
            dimension_semantics=("arbitrary",)),
    )(x, wq, wk, wv)


# speedup vs baseline: 1.8486x; 1.1080x over previous
import functools
import math

import jax
import jax.numpy as jnp
from jax.experimental import pallas as pl
from jax.experimental.pallas import tpu as pltpu


def _fused_kernel(x_ref, wq_ref, wk_ref, wv_ref, o_ref,
                  xb_ref, ta_ref, q_ref, k_ref, v_ref,
                  *, B, T, N, H, d_head, n_k, scale):
    D = x_ref.shape[3]
    R = B * T * N
    S = B * N
    KT = wq_ref.shape[0]
    i = pl.program_id(0)

    @pl.when(i == 0)
    def _pool():
        x = x_ref[...]
        m = jnp.max(x, axis=1, keepdims=True)
        e = jnp.exp(x - m)
        ta = jnp.sum(x * e, axis=1) / jnp.sum(e, axis=1)
        ta_ref[...] = ta.reshape(S, D).astype(jnp.bfloat16)
        xb_ref[...] = x.reshape(R, D).astype(jnp.bfloat16)

    row = pl.multiple_of(i * KT, KT)
    xs = xb_ref[:, pl.ds(row, KT)]
    ts = ta_ref[:, pl.ds(row, KT)]
    wq = wq_ref[...].astype(jnp.bfloat16)
    wk = wk_ref[...].astype(jnp.bfloat16)
    wv = wv_ref[...].astype(jnp.bfloat16)
    pq = jnp.dot(xs, wq, preferred_element_type=jnp.float32)
    pk = jnp.dot(ts, wk, preferred_element_type=jnp.float32)
    pv = jnp.dot(ts, wv, preferred_element_type=jnp.float32)

    @pl.when(i == 0)
    def _init():
        q_ref[...] = pq
        k_ref[...] = pk
        v_ref[...] = pv

    @pl.when(i > 0)
    def _acc():
        q_ref[...] += pq
        k_ref[...] += pk
        v_ref[...] += pv

    @pl.when(i == n_k - 1)
    def _attn():
        q = (q_ref[...] * scale).astype(jnp.bfloat16)
        k = k_ref[...]
        v = v_ref[...]
        C = B * H * N
        rh = (jax.lax.broadcasted_iota(jnp.int32, (C, D), 0) % (H * N)) // N
        lh = jax.lax.broadcasted_iota(jnp.int32, (C, D), 1) // d_head
        hmask = rh == lh
        kb = jnp.broadcast_to(
            k.reshape(B, 1, N, D), (B, H, N, D)).reshape(C, D)
        vb = jnp.broadcast_to(
            v.reshape(B, 1, N, D), (B, H, N, D)).reshape(C, D)
        zero = jnp.zeros((), jnp.bfloat16)
        kbig = jnp.where(hmask, kb.astype(jnp.bfloat16), zero)
        vbig = jnp.where(hmask, vb.astype(jnp.bfloat16), zero)

        s = jax.lax.dot_general(q, kbig, (((1,), (1,)), ((), ())),
                                preferred_element_type=jnp.float32)
        rb = jax.lax.broadcasted_iota(jnp.int32, (R, C), 0) // (T * N)
        cb = jax.lax.broadcasted_iota(jnp.int32, (R, C), 1) // (H * N)
        p = jnp.where(rb == cb, jnp.maximum(s, 0.0),
                      0.0).astype(jnp.bfloat16)

        o = jnp.dot(p, vbig, preferred_element_type=jnp.float32)
        o = o.reshape(B, T, N, D) + v.reshape(B, 1, N, D)
        o_ref[...] = o.astype(o_ref.dtype)


def kernel(x, boxes_in_flat, wq, wk, wv):
    del boxes_in_flat
    B, T, N, D = x.shape
    H = 8
    d_head = D // H
    R = B * T * N
    S = B * N
    scale = 1.0 / math.sqrt(d_head)

    KT = 640
    n_k = D // KT

    kern = functools.partial(
        _fused_kernel, B=B, T=T, N=N, H=H, d_head=d_head, n_k=n_k,
        scale=scale)
    return pl.pallas_call(
        kern,
        out_shape=jax.ShapeDtypeStruct((B, T, N, D), x.dtype),
        grid=(n_k,),
        in_specs=[
            pl.BlockSpec((B, T, N, D), lambda i: (0, 0, 0, 0)),
            pl.BlockSpec((KT, D), lambda i: (i, 0)),
            pl.BlockSpec((KT, D), lambda i: (i, 0)),
            pl.BlockSpec((KT, D), lambda i: (i, 0)),
        ],
        out_specs=pl.BlockSpec((B, T, N, D), lambda i: (0, 0, 0, 0)),
        scratch_shapes=[
            pltpu.VMEM((R, D), jnp.bfloat16),
            pltpu.VMEM((S, D), jnp.bfloat16),
            pltpu.VMEM((R, D), jnp.float32),
            pltpu.VMEM((S, D), jnp.float32),
            pltpu.VMEM((S, D), jnp.float32),
        ],
        compiler_params=pltpu.CompilerParams(
            dimension_semantics=("arbitrary",)),
    )(x, wq, wk, wv)


# speedup vs baseline: 1.8677x; 1.0103x over previous
import functools
import math

import jax
import jax.numpy as jnp
from jax.experimental import pallas as pl
from jax.experimental.pallas import tpu as pltpu


def _fused_kernel(x_ref, wq_ref, wk_ref, wv_ref, o_ref,
                  xb_ref, ta_ref, q_ref, k_ref, v_ref,
                  *, B, T, N, H, d_head, n_k, scale):
    D = x_ref.shape[3]
    R = B * T * N
    S = B * N
    KT = wq_ref.shape[0]
    i = pl.program_id(0)

    @pl.when(i == 0)
    def _pool():
        x = x_ref[...]
        m = jnp.max(x, axis=1, keepdims=True)
        e = jnp.exp(x - m)
        ta = jnp.sum(x * e, axis=1) / jnp.sum(e, axis=1)
        ta_ref[...] = ta.reshape(S, D).astype(jnp.bfloat16)
        xb_ref[...] = x.reshape(R, D).astype(jnp.bfloat16)

    row = pl.multiple_of(i * KT, KT)
    xs = xb_ref[:, pl.ds(row, KT)]
    ts = ta_ref[:, pl.ds(row, KT)]
    wq = wq_ref[...].astype(jnp.bfloat16)
    wk = wk_ref[...].astype(jnp.bfloat16)
    wv = wv_ref[...].astype(jnp.bfloat16)
    pq = jnp.dot(xs, wq, preferred_element_type=jnp.float32)
    pk = jnp.dot(ts, wk, preferred_element_type=jnp.float32)
    pv = jnp.dot(ts, wv, preferred_element_type=jnp.float32)

    @pl.when(i == 0)
    def _init():
        q_ref[...] = pq
        k_ref[...] = pk
        v_ref[...] = pv

    @pl.when(i > 0)
    def _acc():
        q_ref[...] += pq
        k_ref[...] += pk
        v_ref[...] += pv

    @pl.when(i == n_k - 1)
    def _attn():
        q = (q_ref[...] * scale).astype(jnp.bfloat16)
        k = k_ref[...]
        v = v_ref[...]
        C = B * H * N
        rh = (jax.lax.broadcasted_iota(jnp.int32, (C, D), 0) % (H * N)) // N
        lh = jax.lax.broadcasted_iota(jnp.int32, (C, D), 1) // d_head
        hmask = rh == lh
        kb = jnp.broadcast_to(
            k.reshape(B, 1, N, D), (B, H, N, D)).reshape(C, D)
        vb = jnp.broadcast_to(
            v.reshape(B, 1, N, D), (B, H, N, D)).reshape(C, D)
        zero = jnp.zeros((), jnp.bfloat16)
        kbig = jnp.where(hmask, kb.astype(jnp.bfloat16), zero)
        vbig = jnp.where(hmask, vb.astype(jnp.bfloat16), zero)

        s = jax.lax.dot_general(q, kbig, (((1,), (1,)), ((), ())),
                                preferred_element_type=jnp.float32)
        rb = jax.lax.broadcasted_iota(jnp.int32, (R, C), 0) // (T * N)
        cb = jax.lax.broadcasted_iota(jnp.int32, (R, C), 1) // (H * N)
        p = jnp.where(rb == cb, jnp.maximum(s, 0.0),
                      0.0).astype(jnp.bfloat16)

        o = jnp.dot(p, vbig, preferred_element_type=jnp.float32)
        o = o.reshape(B, T, N, D) + v.reshape(B, 1, N, D)
        o_ref[...] = o.astype(o_ref.dtype)


def kernel(x, boxes_in_flat, wq, wk, wv):
    del boxes_in_flat
    B, T, N, D = x.shape
    H = 8
    d_head = D // H
    R = B * T * N
    S = B * N
    scale = 1.0 / math.sqrt(d_head)

    KT = 1280
    n_k = D // KT

    kern = functools.partial(
        _fused_kernel, B=B, T=T, N=N, H=H, d_head=d_head, n_k=n_k,
        scale=scale)
    return pl.pallas_call(
        kern,
        out_shape=jax.ShapeDtypeStruct((B, T, N, D), x.dtype),
        grid=(n_k,),
        in_specs=[
            pl.BlockSpec((B, T, N, D), lambda i: (0, 0, 0, 0)),
            pl.BlockSpec((KT, D), lambda i: (i, 0)),
            pl.BlockSpec((KT, D), lambda i: (i, 0)),
            pl.BlockSpec((KT, D), lambda i: (i, 0)),
        ],
        out_specs=pl.BlockSpec((B, T, N, D), lambda i: (0, 0, 0, 0)),
        scratch_shapes=[
            pltpu.VMEM((R, D), jnp.bfloat16),
            pltpu.VMEM((S, D), jnp.bfloat16),
            pltpu.VMEM((R, D), jnp.float32),
            pltpu.VMEM((S, D), jnp.float32),
            pltpu.VMEM((S, D), jnp.float32),
        ],
        compiler_params=pltpu.CompilerParams(
            dimension_semantics=("arbitrary",)),
    )(x, wq, wk, wv)
